# Initial kernel scaffold; baseline (speedup 1.0000x reference)
import jax
import jax.numpy as jnp
from jax.experimental import pallas as pl


def kernel(pos, mapping, mapping_batch, atom_types, theta_0, k_0, theta_1, k_1, theta_2, k_2):
    raise NotImplementedError("write your pallas kernel here")


# speedup vs baseline: 143.7421x; 143.7421x over previous
import functools

import jax
import jax.numpy as jnp
from jax import lax
from jax.experimental import pallas as pl
from jax.experimental.pallas import tpu as pltpu
from jax.experimental.pallas import tpu_sc as plsc

N_NODES = 100000
N_DIH = 1600000
N_TYPES = 20
N_BATCH = 1000

NC = 2
NS = 16
NW = NC * NS
L = 16

CHUNK = 512
NSUB = CHUNK // 128
N_BLOCKS = N_DIH // 128
BLOCKS_PER_TILE = N_BLOCKS // NW
PER_TILE = BLOCKS_PER_TILE * 128
FULL_CHUNKS = PER_TILE // CHUNK
TAIL_NSUB = (PER_TILE - FULL_CHUNKS * CHUNK) // 128
EXTRA_BASE = PER_TILE * NW
EXTRA_BLOCKS = N_BLOCKS - BLOCKS_PER_TILE * NW

ACC_SLOTS = 1024
ROW = 16

_f32 = jnp.float32
_i32 = jnp.int32


def _spl_f(v):
    return jnp.full((L,), v, _f32)


def _spl_i(v):
    return jnp.full((L,), v, _i32)


def _rsqrt16(x):
    xi = plsc.bitcast(x, _i32)
    yi = _spl_i(0x5F3759DF) - (xi >> 1)
    y = plsc.bitcast(yi, _f32)
    half_x = _spl_f(0.5) * x
    for _ in range(3):
        y = y * (_spl_f(1.5) - half_x * y * y)
    return y


def _sc_body(nodes, mapp, batp, ptab, out,
             map_v, bat_v, pos_v, pidx_v, cos_v, sin_v, par_v, acc_v, eng_v,
             sem):
    cid = lax.axis_index("c")
    sid = lax.axis_index("s")
    wid = sid * NC + cid
    tile_base = wid * PER_TILE

    lane = lax.iota(_i32, L)

    def _zero(i, carry):
        acc_v[pl.ds(i * L, L)] = _spl_f(0.0)
        return carry
    lax.fori_loop(0, (L * ACC_SLOTS) // L, _zero, 0)

    def _pass_a(kk):
        def body(g8, carry):
            off = g8 * L
            row = lane + off

            def ld(j, c):
                return plsc.load_gather(
                    pos_v, [_spl_i(j), _spl_i(kk), row, _spl_i(c)])

            p = [[ld(j, c) for c in range(3)] for j in range(4)]
            ti = [ld(j, 3).astype(_i32) for j in range(4)]

            b0 = [p[0][c] - p[1][c] for c in range(3)]
            b1 = [p[2][c] - p[1][c] for c in range(3)]
            b2 = [p[3][c] - p[2][c] for c in range(3)]
            s = b1[0] * b1[0] + b1[1] * b1[1] + b1[2] * b1[2]
            d01 = b0[0] * b1[0] + b0[1] * b1[1] + b0[2] * b1[2]
            d21 = b2[0] * b1[0] + b2[1] * b1[1] + b2[2] * b1[2]
            d02 = b0[0] * b2[0] + b0[1] * b2[1] + b0[2] * b2[2]
            crx = b0[1] * b2[2] - b0[2] * b2[1]
            cry = b0[2] * b2[0] - b0[0] * b2[2]
            crz = b0[0] * b2[1] - b0[1] * b2[0]
            tt = b1[0] * crx + b1[1] * cry + b1[2] * crz

            zero = _spl_f(0.0)
            one = _spl_f(1.0)
            szero = s == zero
            rs = _rsqrt16(jnp.where(szero, one, s))
            x = s * d02 - d01 * d21
            y = s * rs * tt
            x = jnp.where(szero, d02, x)
            y = jnp.where(szero, zero, y)
            r2 = x * x + y * y
            r2z = r2 == zero
            inv = _rsqrt16(jnp.where(r2z, one, r2))
            cosv = jnp.where(r2z, one, x * inv)
            sinv = jnp.where(r2z, zero, y * inv)

            goff = kk * 128 + off
            cos_v[pl.ds(goff, L)] = cosv
            sin_v[pl.ds(goff, L)] = sinv

            twenty = _spl_i(N_TYPES)
            pidx = ((ti[0] * twenty + ti[1]) * twenty + ti[2]) * twenty + ti[3]
            pidx_v[kk, pl.ds(off, L)] = pidx
            return carry
        return body

    def _pass_b(kk):
        def body(g8, carry):
            off = g8 * L
            goff = kk * 128 + off
            row = lane + off
            pr = [plsc.load_gather(par_v, [_spl_i(kk), row, _spl_i(c)])
                  for c in range(7)]
            cosv = cos_v[pl.ds(goff, L)]
            sinv = sin_v[pl.ds(goff, L)]
            bid = bat_v[pl.ds(goff, L)]
            one = _spl_f(1.0)
            two = _spl_f(2.0)
            c2 = two * cosv * cosv - one
            s2 = two * sinv * cosv
            dd = two * c2
            c3 = cosv * (dd - one)
            s3 = sinv * (dd + one)
            v = pr[0] - (pr[1] * cosv + pr[2] * sinv + pr[3] * c2 +
                         pr[4] * s2 + pr[5] * c3 + pr[6] * s3)
            addr = lane * _spl_i(ACC_SLOTS) + bid
            plsc.addupdate_scatter(acc_v, [addr], v)
            return carry
        return body

    def emit_chunk(base, nsub):
        width = nsub * 128
        pltpu.sync_copy(mapp.at[:, pl.ds(base, width)],
                        map_v.at[:, pl.ds(0, width)])
        pltpu.sync_copy(batp.at[pl.ds(base, width)],
                        bat_v.at[pl.ds(0, width)])
        cps = [pltpu.async_copy(
                   nodes.at[map_v.at[j, pl.ds(kk * 128, 128)]],
                   pos_v.at[j, kk], sem)
               for j in range(4) for kk in range(nsub)]
        for cp in cps:
            cp.wait()
        for kk in range(nsub):
            lax.fori_loop(0, 128 // L, _pass_a(kk), 0)
        pcs = [pltpu.async_copy(ptab.at[pidx_v.at[kk]], par_v.at[kk], sem)
               for kk in range(nsub)]
        for cp in pcs:
            cp.wait()
        for kk in range(nsub):
            lax.fori_loop(0, 128 // L, _pass_b(kk), 0)

    def chunk_loop(ci, carry):
        emit_chunk(tile_base + ci * CHUNK, NSUB)
        return carry
    lax.fori_loop(0, FULL_CHUNKS, chunk_loop, 0)
    emit_chunk(tile_base + FULL_CHUNKS * CHUNK, TAIL_NSUB)

    @pl.when(wid < EXTRA_BLOCKS)
    def _extra():
        emit_chunk(EXTRA_BASE + wid * 128, 1)

    def _fold(j, carry):
        col = j * L
        v = acc_v[pl.ds(col, L)]
        for r in range(1, L):
            v = v + acc_v[pl.ds(r * ACC_SLOTS + col, L)]
        eng_v[pl.ds(col, L)] = v
        return carry
    lax.fori_loop(0, ACC_SLOTS // L, _fold, 0)

    pltpu.sync_copy(eng_v, out.at[pl.ds(wid * ACC_SLOTS, ACC_SLOTS)])


_sc_kernel = pl.kernel(
    _sc_body,
    out_type=jax.ShapeDtypeStruct((NW * ACC_SLOTS,), _f32),
    mesh=plsc.VectorSubcoreMesh(core_axis_name="c", subcore_axis_name="s"),
    compiler_params=pltpu.CompilerParams(
        needs_layout_passes=False, use_tc_tiling_on_sc=False),
    scratch_types=[
        pltpu.VMEM((4, CHUNK), _i32),
        pltpu.VMEM((CHUNK,), _i32),
        pltpu.VMEM((4, NSUB, 128, ROW), _f32),
        pltpu.VMEM((NSUB, 128), _i32),
        pltpu.VMEM((CHUNK,), _f32),
        pltpu.VMEM((CHUNK,), _f32),
        pltpu.VMEM((NSUB, 128, ROW), _f32),
        pltpu.VMEM((L * ACC_SLOTS,), _f32),
        pltpu.VMEM((ACC_SLOTS,), _f32),
        pltpu.SemaphoreType.DMA,
    ],
)


def _prep_body(t0, k0, t1, k1, t2, k2, csum, a0, b0, a1, b1, a2, b2):
    csum[...] = k0[...] + k1[...] + k2[...]
    a0[...] = k0[...] * jnp.cos(t0[...])
    b0[...] = k0[...] * jnp.sin(t0[...])
    a1[...] = k1[...] * jnp.cos(t1[...])
    b1[...] = k1[...] * jnp.sin(t1[...])
    a2[...] = k2[...] * jnp.cos(t2[...])
    b2[...] = k2[...] * jnp.sin(t2[...])


def _finish_body(x, o):
    o[...] = jnp.sum(x[...], axis=0)


def kernel(pos, mapping, mapping_batch, atom_types,
           theta_0, k_0, theta_1, k_1, theta_2, k_2):
    ntab = N_TYPES ** 4
    shape2d = (ntab // 128, 128)
    tabs = [a.reshape(shape2d) for a in
            (theta_0, k_0, theta_1, k_1, theta_2, k_2)]
    coef = pl.pallas_call(
        _prep_body,
        out_shape=[jax.ShapeDtypeStruct(shape2d, _f32)] * 7,
    )(*tabs)
    ptab = jnp.concatenate(
        [c.reshape(-1, 1) for c in coef] +
        [jnp.zeros((ntab, ROW - 7), _f32)], axis=1)

    nodes = jnp.concatenate(
        [pos, atom_types.astype(_f32)[:, None],
         jnp.zeros((N_NODES, ROW - 4), _f32)], axis=1)

    part = _sc_kernel(nodes, mapping, mapping_batch, ptab)

    eng = pl.pallas_call(
        _finish_body,
        out_shape=jax.ShapeDtypeStruct((ACC_SLOTS // 128, 128), _f32),
    )(part.reshape(NW, ACC_SLOTS // 128, 128))
    return eng.reshape(ACC_SLOTS)[:N_BATCH]


# speedup vs baseline: 156.4214x; 1.0882x over previous
import functools

import jax
import jax.numpy as jnp
from jax import lax
from jax.experimental import pallas as pl
from jax.experimental.pallas import tpu as pltpu
from jax.experimental.pallas import tpu_sc as plsc

N_NODES = 100000
N_DIH = 1600000
N_TYPES = 20
N_BATCH = 1000

NC = 2
NS = 16
NW = NC * NS
L = 16

CHUNK = 512
NSUB = CHUNK // 128
N_BLOCKS = N_DIH // 128
BLOCKS_PER_TILE = N_BLOCKS // NW
PER_TILE = BLOCKS_PER_TILE * 128
FULL_CHUNKS = PER_TILE // CHUNK
TAIL_NSUB = (PER_TILE - FULL_CHUNKS * CHUNK) // 128
EXTRA_BASE = PER_TILE * NW
EXTRA_BLOCKS = N_BLOCKS - BLOCKS_PER_TILE * NW

ACC_SLOTS = 1024
ROW = 16

_f32 = jnp.float32
_i32 = jnp.int32


def _spl_f(v):
    return jnp.full((L,), v, _f32)


def _spl_i(v):
    return jnp.full((L,), v, _i32)


def _rsqrt16(x):
    xi = plsc.bitcast(x, _i32)
    yi = _spl_i(0x5F3759DF) - (xi >> 1)
    y = plsc.bitcast(yi, _f32)
    half_x = _spl_f(0.5) * x
    for _ in range(3):
        y = y * (_spl_f(1.5) - half_x * y * y)
    return y


def _sc_body(nodes, mapp, batp, ptab, out,
             map_v, bat_v, pos_v, pidx_v, cos_v, sin_v, par_v, acc_v, eng_v,
             sem):
    cid = lax.axis_index("c")
    sid = lax.axis_index("s")
    wid = sid * NC + cid
    tile_base = wid * PER_TILE

    lane = lax.iota(_i32, L)

    def _zero(i, carry):
        acc_v[pl.ds(i * L, L)] = _spl_f(0.0)
        return carry
    lax.fori_loop(0, (L * ACC_SLOTS) // L, _zero, 0)

    def _pass_a(kk):
        def body(g8, carry):
            off = g8 * L
            row = lane + off

            def ld(j, c):
                return plsc.load_gather(
                    pos_v, [_spl_i(j), _spl_i(kk), row, _spl_i(c)])

            p = [[ld(j, c) for c in range(3)] for j in range(4)]
            ti = [ld(j, 3).astype(_i32) for j in range(4)]

            b0 = [p[0][c] - p[1][c] for c in range(3)]
            b1 = [p[2][c] - p[1][c] for c in range(3)]
            b2 = [p[3][c] - p[2][c] for c in range(3)]
            s = b1[0] * b1[0] + b1[1] * b1[1] + b1[2] * b1[2]
            d01 = b0[0] * b1[0] + b0[1] * b1[1] + b0[2] * b1[2]
            d21 = b2[0] * b1[0] + b2[1] * b1[1] + b2[2] * b1[2]
            d02 = b0[0] * b2[0] + b0[1] * b2[1] + b0[2] * b2[2]
            crx = b0[1] * b2[2] - b0[2] * b2[1]
            cry = b0[2] * b2[0] - b0[0] * b2[2]
            crz = b0[0] * b2[1] - b0[1] * b2[0]
            tt = b1[0] * crx + b1[1] * cry + b1[2] * crz

            zero = _spl_f(0.0)
            one = _spl_f(1.0)
            szero = s == zero
            rs = _rsqrt16(jnp.where(szero, one, s))
            x = s * d02 - d01 * d21
            y = s * rs * tt
            x = jnp.where(szero, d02, x)
            y = jnp.where(szero, zero, y)
            r2 = x * x + y * y
            r2z = r2 == zero
            inv = _rsqrt16(jnp.where(r2z, one, r2))
            cosv = jnp.where(r2z, one, x * inv)
            sinv = jnp.where(r2z, zero, y * inv)

            goff = kk * 128 + off
            cos_v[pl.ds(goff, L)] = cosv
            sin_v[pl.ds(goff, L)] = sinv

            twenty = _spl_i(N_TYPES)
            pidx = ((ti[0] * twenty + ti[1]) * twenty + ti[2]) * twenty + ti[3]
            pidx_v[kk, pl.ds(off, L)] = pidx
            return carry
        return body

    def _pass_b(kk):
        def body(g8, carry):
            off = g8 * L
            goff = kk * 128 + off
            row = lane + off
            pr = [plsc.load_gather(par_v, [_spl_i(kk), row, _spl_i(c)])
                  for c in range(7)]
            cosv = cos_v[pl.ds(goff, L)]
            sinv = sin_v[pl.ds(goff, L)]
            bid = bat_v[pl.ds(goff, L)]
            one = _spl_f(1.0)
            two = _spl_f(2.0)
            c2 = two * cosv * cosv - one
            s2 = two * sinv * cosv
            dd = two * c2
            c3 = cosv * (dd - one)
            s3 = sinv * (dd + one)
            v = pr[0] - (pr[1] * cosv + pr[2] * sinv + pr[3] * c2 +
                         pr[4] * s2 + pr[5] * c3 + pr[6] * s3)
            addr = lane * _spl_i(ACC_SLOTS) + bid
            plsc.addupdate_scatter(acc_v, [addr], v)
            return carry
        return body

    def emit_chunk(base, nsub):
        width = nsub * 128
        pltpu.sync_copy(mapp.at[:, pl.ds(base, width)],
                        map_v.at[:, pl.ds(0, width)])
        pltpu.sync_copy(batp.at[pl.ds(base, width)],
                        bat_v.at[pl.ds(0, width)])
        cps = [pltpu.async_copy(
                   nodes.at[map_v.at[j, pl.ds(kk * 128, 128)]],
                   pos_v.at[j, kk], sem)
               for j in range(4) for kk in range(nsub)]
        for cp in cps:
            cp.wait()
        for kk in range(nsub):
            lax.fori_loop(0, 128 // L, _pass_a(kk), 0)
        pcs = [pltpu.async_copy(ptab.at[pidx_v.at[kk]], par_v.at[kk], sem)
               for kk in range(nsub)]
        for cp in pcs:
            cp.wait()
        for kk in range(nsub):
            lax.fori_loop(0, 128 // L, _pass_b(kk), 0)

    def chunk_loop(ci, carry):
        emit_chunk(tile_base + ci * CHUNK, NSUB)
        return carry
    lax.fori_loop(0, FULL_CHUNKS, chunk_loop, 0)
    emit_chunk(tile_base + FULL_CHUNKS * CHUNK, TAIL_NSUB)

    @pl.when(wid < EXTRA_BLOCKS)
    def _extra():
        emit_chunk(EXTRA_BASE + wid * 128, 1)

    def _fold(j, carry):
        col = j * L
        v = acc_v[pl.ds(col, L)]
        for r in range(1, L):
            v = v + acc_v[pl.ds(r * ACC_SLOTS + col, L)]
        eng_v[pl.ds(col, L)] = v
        return carry
    lax.fori_loop(0, ACC_SLOTS // L, _fold, 0)

    pltpu.sync_copy(eng_v, out.at[pl.ds(wid * ACC_SLOTS, ACC_SLOTS)])


_sc_kernel = pl.kernel(
    _sc_body,
    out_type=jax.ShapeDtypeStruct((NW * ACC_SLOTS,), _f32),
    mesh=plsc.VectorSubcoreMesh(core_axis_name="c", subcore_axis_name="s"),
    compiler_params=pltpu.CompilerParams(
        needs_layout_passes=False, use_tc_tiling_on_sc=False),
    scratch_types=[
        pltpu.VMEM((4, CHUNK), _i32),
        pltpu.VMEM((CHUNK,), _i32),
        pltpu.VMEM((4, NSUB, 128, ROW), _f32),
        pltpu.VMEM((NSUB, 128), _i32),
        pltpu.VMEM((CHUNK,), _f32),
        pltpu.VMEM((CHUNK,), _f32),
        pltpu.VMEM((NSUB, 128, ROW), _f32),
        pltpu.VMEM((L * ACC_SLOTS,), _f32),
        pltpu.VMEM((ACC_SLOTS,), _f32),
        pltpu.SemaphoreType.DMA,
    ],
)


PK_N_CNT = 3136
PK_N_STRIDE = 3128
PK_P_CNT = 5008
PK_P_STRIDE = 5000
PK_P_HALF = (2512, 2496)


def _pack_body(posf, typ, c0, c1, c2, c3, c4, c5, c6,
               nodes_out, ptab_out,
               posb, typb, packb, cb0, cb1, cb2, cb3, cb4, cb5, cb6, packp,
               sem):
    cid = lax.axis_index("c")
    sid = lax.axis_index("s")
    wid = sid * NC + cid
    lane = lax.iota(_i32, L)

    nbase = jnp.minimum(wid * PK_N_STRIDE, N_NODES - PK_N_CNT)
    pltpu.sync_copy(posf.at[pl.ds(nbase * 3, PK_N_CNT * 3)], posb)
    pltpu.sync_copy(typ.at[pl.ds(nbase, PK_N_CNT)], typb)
    for h in range(2):
        hoff = h * (PK_N_CNT // 2)

        def ngrp(gl, carry):
            lh = lane + gl * L
            src = (lh + hoff) * _spl_i(3)
            dst = lh * _spl_i(ROW)
            for c in range(3):
                v = plsc.load_gather(posb, [src + _spl_i(c)])
                plsc.store_scatter(packb, [dst + _spl_i(c)], v)
            tv = typb[pl.ds(hoff + gl * L, L)].astype(_f32)
            plsc.store_scatter(packb, [dst + _spl_i(3)], tv)
            return carry
        lax.fori_loop(0, PK_N_CNT // 2 // L, ngrp, 0)
        pltpu.sync_copy(
            packb, nodes_out.at[pl.ds((nbase + hoff) * ROW, PK_N_CNT // 2 * ROW)])

    pbase = jnp.minimum(wid * PK_P_STRIDE, N_TYPES ** 4 - PK_P_CNT)
    cbs = [cb0, cb1, cb2, cb3, cb4, cb5, cb6]
    planes = [c0, c1, c2, c3, c4, c5, c6]
    for h in range(2):
        hoff = h * PK_P_HALF[0]
        cnt = PK_P_HALF[h]
        for cb, pf in zip(cbs, planes):
            pltpu.sync_copy(pf.at[pl.ds(pbase + hoff, cnt)],
                            cb.at[pl.ds(0, cnt)])

        def pgrp(gl, carry):
            lh = lane + gl * L
            dst = lh * _spl_i(ROW)
            for c in range(7):
                v = cbs[c][pl.ds(gl * L, L)]
                plsc.store_scatter(packp, [dst + _spl_i(c)], v)
            return carry
        lax.fori_loop(0, cnt // L, pgrp, 0)
        pltpu.sync_copy(packp.at[pl.ds(0, cnt * ROW)],
                        ptab_out.at[pl.ds((pbase + hoff) * ROW, cnt * ROW)])


_pack_kernel = pl.kernel(
    _pack_body,
    out_type=(jax.ShapeDtypeStruct((N_NODES * ROW,), _f32),
              jax.ShapeDtypeStruct((N_TYPES ** 4 * ROW,), _f32)),
    mesh=plsc.VectorSubcoreMesh(core_axis_name="c", subcore_axis_name="s"),
    compiler_params=pltpu.CompilerParams(
        needs_layout_passes=False, use_tc_tiling_on_sc=False),
    scratch_types=[
        pltpu.VMEM((PK_N_CNT * 3,), _f32),
        pltpu.VMEM((PK_N_CNT,), _i32),
        pltpu.VMEM((PK_N_CNT // 2 * ROW,), _f32),
    ] + [pltpu.VMEM((PK_P_HALF[0],), _f32)] * 7
    + [
        pltpu.VMEM((PK_P_HALF[0] * ROW,), _f32),
        pltpu.SemaphoreType.DMA,
    ],
)


def _prep_body(t0, k0, t1, k1, t2, k2, csum, a0, b0, a1, b1, a2, b2):
    csum[...] = k0[...] + k1[...] + k2[...]
    a0[...] = k0[...] * jnp.cos(t0[...])
    b0[...] = k0[...] * jnp.sin(t0[...])
    a1[...] = k1[...] * jnp.cos(t1[...])
    b1[...] = k1[...] * jnp.sin(t1[...])
    a2[...] = k2[...] * jnp.cos(t2[...])
    b2[...] = k2[...] * jnp.sin(t2[...])


def _finish_body(x, o):
    o[...] = jnp.sum(x[...], axis=0)


def kernel(pos, mapping, mapping_batch, atom_types,
           theta_0, k_0, theta_1, k_1, theta_2, k_2):
    ntab = N_TYPES ** 4
    shape2d = (ntab // 128, 128)
    tabs = [a.reshape(shape2d) for a in
            (theta_0, k_0, theta_1, k_1, theta_2, k_2)]
    coef = pl.pallas_call(
        _prep_body,
        out_shape=[jax.ShapeDtypeStruct(shape2d, _f32)] * 7,
    )(*tabs)

    nodes_f, ptab_f = _pack_kernel(
        pos.reshape(-1), atom_types, *[c.reshape(-1) for c in coef])
    nodes = nodes_f.reshape(N_NODES, ROW)
    ptab = ptab_f.reshape(ntab, ROW)

    part = _sc_kernel(nodes, mapping, mapping_batch, ptab)

    eng = pl.pallas_call(
        _finish_body,
        out_shape=jax.ShapeDtypeStruct((ACC_SLOTS // 128, 128), _f32),
    )(part.reshape(NW, ACC_SLOTS // 128, 128))
    return eng.reshape(ACC_SLOTS)[:N_BATCH]


# speedup vs baseline: 186.3745x; 1.1915x over previous
import functools

import jax
import jax.numpy as jnp
from jax import lax
from jax.experimental import pallas as pl
from jax.experimental.pallas import tpu as pltpu
from jax.experimental.pallas import tpu_sc as plsc

N_NODES = 100000
N_DIH = 1600000
N_TYPES = 20
N_BATCH = 1000

NC = 2
NS = 16
NW = NC * NS
L = 16

CHUNK = 512
NSUB = CHUNK // 128
N_BLOCKS = N_DIH // 128
BLOCKS_PER_TILE = N_BLOCKS // NW
PER_TILE = BLOCKS_PER_TILE * 128
FULL_CHUNKS = PER_TILE // CHUNK
TAIL_NSUB = (PER_TILE - FULL_CHUNKS * CHUNK) // 128
EXTRA_BASE = PER_TILE * NW
EXTRA_BLOCKS = N_BLOCKS - BLOCKS_PER_TILE * NW

ACC_SLOTS = 1024
ROW = 16

_f32 = jnp.float32
_i32 = jnp.int32


def _spl_f(v):
    return jnp.full((L,), v, _f32)


def _spl_i(v):
    return jnp.full((L,), v, _i32)


def _rsqrt16(x):
    xi = plsc.bitcast(x, _i32)
    yi = _spl_i(0x5F3759DF) - (xi >> 1)
    y = plsc.bitcast(yi, _f32)
    half_x = _spl_f(0.5) * x
    for _ in range(3):
        y = y * (_spl_f(1.5) - half_x * y * y)
    return y


def _sc_body(nodes, m0, m1, m2, m3, batp, ptab, out,
             map_v, bat_v, pos_v, pidx_v, cos_v, sin_v, par_v, acc_v, eng_v,
             sem):
    mrows = (m0, m1, m2, m3)
    cid = lax.axis_index("c")
    sid = lax.axis_index("s")
    wid = sid * NC + cid
    tile_base = wid * PER_TILE

    lane = lax.iota(_i32, L)

    def _zero(i, carry):
        acc_v[pl.ds(i * L, L)] = _spl_f(0.0)
        return carry
    lax.fori_loop(0, (L * ACC_SLOTS) // L, _zero, 0)

    def _pass_a(kk):
        def body(g8, carry):
            off = g8 * L
            row = lane + off

            def ld(j, c):
                return plsc.load_gather(
                    pos_v, [_spl_i(j), _spl_i(kk), row, _spl_i(c)])

            p = [[ld(j, c) for c in range(3)] for j in range(4)]
            ti = [ld(j, 3).astype(_i32) for j in range(4)]

            b0 = [p[0][c] - p[1][c] for c in range(3)]
            b1 = [p[2][c] - p[1][c] for c in range(3)]
            b2 = [p[3][c] - p[2][c] for c in range(3)]
            s = b1[0] * b1[0] + b1[1] * b1[1] + b1[2] * b1[2]
            d01 = b0[0] * b1[0] + b0[1] * b1[1] + b0[2] * b1[2]
            d21 = b2[0] * b1[0] + b2[1] * b1[1] + b2[2] * b1[2]
            d02 = b0[0] * b2[0] + b0[1] * b2[1] + b0[2] * b2[2]
            crx = b0[1] * b2[2] - b0[2] * b2[1]
            cry = b0[2] * b2[0] - b0[0] * b2[2]
            crz = b0[0] * b2[1] - b0[1] * b2[0]
            tt = b1[0] * crx + b1[1] * cry + b1[2] * crz

            zero = _spl_f(0.0)
            one = _spl_f(1.0)
            szero = s == zero
            rs = _rsqrt16(jnp.where(szero, one, s))
            x = s * d02 - d01 * d21
            y = s * rs * tt
            x = jnp.where(szero, d02, x)
            y = jnp.where(szero, zero, y)
            r2 = x * x + y * y
            r2z = r2 == zero
            inv = _rsqrt16(jnp.where(r2z, one, r2))
            cosv = jnp.where(r2z, one, x * inv)
            sinv = jnp.where(r2z, zero, y * inv)

            goff = kk * 128 + off
            cos_v[pl.ds(goff, L)] = cosv
            sin_v[pl.ds(goff, L)] = sinv

            twenty = _spl_i(N_TYPES)
            pidx = ((ti[0] * twenty + ti[1]) * twenty + ti[2]) * twenty + ti[3]
            pidx_v[kk, pl.ds(off, L)] = pidx
            return carry
        return body

    def _pass_b(kk):
        def body(g8, carry):
            off = g8 * L
            goff = kk * 128 + off
            row = lane + off
            pr = [plsc.load_gather(par_v, [_spl_i(kk), row, _spl_i(c)])
                  for c in range(7)]
            cosv = cos_v[pl.ds(goff, L)]
            sinv = sin_v[pl.ds(goff, L)]
            bid = bat_v[pl.ds(goff, L)]
            one = _spl_f(1.0)
            two = _spl_f(2.0)
            c2 = two * cosv * cosv - one
            s2 = two * sinv * cosv
            dd = two * c2
            c3 = cosv * (dd - one)
            s3 = sinv * (dd + one)
            v = pr[0] - (pr[1] * cosv + pr[2] * sinv + pr[3] * c2 +
                         pr[4] * s2 + pr[5] * c3 + pr[6] * s3)
            addr = lane * _spl_i(ACC_SLOTS) + bid
            plsc.addupdate_scatter(acc_v, [addr], v)
            return carry
        return body

    def emit_chunk(base, nsub):
        width = nsub * 128
        for j in range(4):
            pltpu.sync_copy(mrows[j].at[pl.ds(base, width)],
                            map_v.at[j, pl.ds(0, width)])
        pltpu.sync_copy(batp.at[pl.ds(base, width)],
                        bat_v.at[pl.ds(0, width)])
        cps = [pltpu.async_copy(
                   nodes.at[map_v.at[j, pl.ds(kk * 128, 128)]],
                   pos_v.at[j, kk], sem)
               for j in range(4) for kk in range(nsub)]
        for cp in cps:
            cp.wait()
        for kk in range(nsub):
            lax.fori_loop(0, 128 // L, _pass_a(kk), 0)
        pcs = [pltpu.async_copy(ptab.at[pidx_v.at[kk]], par_v.at[kk], sem)
               for kk in range(nsub)]
        for cp in pcs:
            cp.wait()
        for kk in range(nsub):
            lax.fori_loop(0, 128 // L, _pass_b(kk), 0)

    def chunk_loop(ci, carry):
        emit_chunk(tile_base + ci * CHUNK, NSUB)
        return carry
    lax.fori_loop(0, FULL_CHUNKS, chunk_loop, 0)
    emit_chunk(tile_base + FULL_CHUNKS * CHUNK, TAIL_NSUB)

    @pl.when(wid < EXTRA_BLOCKS)
    def _extra():
        emit_chunk(EXTRA_BASE + wid * 128, 1)

    def _fold(j, carry):
        col = j * L
        v = acc_v[pl.ds(col, L)]
        for r in range(1, L):
            v = v + acc_v[pl.ds(r * ACC_SLOTS + col, L)]
        eng_v[pl.ds(col, L)] = v
        return carry
    lax.fori_loop(0, ACC_SLOTS // L, _fold, 0)

    pltpu.sync_copy(eng_v, out.at[pl.ds(wid * ACC_SLOTS, ACC_SLOTS)])


_sc_kernel = pl.kernel(
    _sc_body,
    out_type=jax.ShapeDtypeStruct((NW * ACC_SLOTS,), _f32),
    mesh=plsc.VectorSubcoreMesh(core_axis_name="c", subcore_axis_name="s"),
    compiler_params=pltpu.CompilerParams(
        needs_layout_passes=False, use_tc_tiling_on_sc=False),
    scratch_types=[
        pltpu.VMEM((4, CHUNK), _i32),
        pltpu.VMEM((CHUNK,), _i32),
        pltpu.VMEM((4, NSUB, 128, ROW), _f32),
        pltpu.VMEM((NSUB, 128), _i32),
        pltpu.VMEM((CHUNK,), _f32),
        pltpu.VMEM((CHUNK,), _f32),
        pltpu.VMEM((NSUB, 128, ROW), _f32),
        pltpu.VMEM((L * ACC_SLOTS,), _f32),
        pltpu.VMEM((ACC_SLOTS,), _f32),
        pltpu.SemaphoreType.DMA,
    ],
)


PK_N_CNT = 3136
PK_N_STRIDE = 3128
PK_P_CNT = 5008
PK_P_STRIDE = 5000
PK_P_HALF = (2512, 2496)


def _pack_body(posf, typ, c0, c1, c2, c3, c4, c5, c6,
               nodes_out, ptab_out,
               posb, typb, packb, cb0, cb1, cb2, cb3, cb4, cb5, cb6, packp,
               sem):
    cid = lax.axis_index("c")
    sid = lax.axis_index("s")
    wid = sid * NC + cid
    lane = lax.iota(_i32, L)

    nbase = jnp.minimum(wid * PK_N_STRIDE, N_NODES - PK_N_CNT)
    pltpu.sync_copy(posf.at[pl.ds(nbase * 3, PK_N_CNT * 3)], posb)
    pltpu.sync_copy(typ.at[pl.ds(nbase, PK_N_CNT)], typb)
    for h in range(2):
        hoff = h * (PK_N_CNT // 2)

        def ngrp(gl, carry):
            lh = lane + gl * L
            src = (lh + hoff) * _spl_i(3)
            dst = lh * _spl_i(ROW)
            for c in range(3):
                v = plsc.load_gather(posb, [src + _spl_i(c)])
                plsc.store_scatter(packb, [dst + _spl_i(c)], v)
            tv = typb[pl.ds(hoff + gl * L, L)].astype(_f32)
            plsc.store_scatter(packb, [dst + _spl_i(3)], tv)
            return carry
        lax.fori_loop(0, PK_N_CNT // 2 // L, ngrp, 0)
        pltpu.sync_copy(
            packb, nodes_out.at[pl.ds((nbase + hoff) * ROW, PK_N_CNT // 2 * ROW)])

    pbase = jnp.minimum(wid * PK_P_STRIDE, N_TYPES ** 4 - PK_P_CNT)
    cbs = [cb0, cb1, cb2, cb3, cb4, cb5, cb6]
    planes = [c0, c1, c2, c3, c4, c5, c6]
    for h in range(2):
        hoff = h * PK_P_HALF[0]
        cnt = PK_P_HALF[h]
        for cb, pf in zip(cbs, planes):
            pltpu.sync_copy(pf.at[pl.ds(pbase + hoff, cnt)],
                            cb.at[pl.ds(0, cnt)])

        def pgrp(gl, carry):
            lh = lane + gl * L
            dst = lh * _spl_i(ROW)
            for c in range(7):
                v = cbs[c][pl.ds(gl * L, L)]
                plsc.store_scatter(packp, [dst + _spl_i(c)], v)
            return carry
        lax.fori_loop(0, cnt // L, pgrp, 0)
        pltpu.sync_copy(packp.at[pl.ds(0, cnt * ROW)],
                        ptab_out.at[pl.ds((pbase + hoff) * ROW, cnt * ROW)])


_pack_kernel = pl.kernel(
    _pack_body,
    out_type=(jax.ShapeDtypeStruct((N_NODES * ROW,), _f32),
              jax.ShapeDtypeStruct((N_TYPES ** 4 * ROW,), _f32)),
    mesh=plsc.VectorSubcoreMesh(core_axis_name="c", subcore_axis_name="s"),
    compiler_params=pltpu.CompilerParams(
        needs_layout_passes=False, use_tc_tiling_on_sc=False),
    scratch_types=[
        pltpu.VMEM((PK_N_CNT * 3,), _f32),
        pltpu.VMEM((PK_N_CNT,), _i32),
        pltpu.VMEM((PK_N_CNT // 2 * ROW,), _f32),
    ] + [pltpu.VMEM((PK_P_HALF[0],), _f32)] * 7
    + [
        pltpu.VMEM((PK_P_HALF[0] * ROW,), _f32),
        pltpu.SemaphoreType.DMA,
    ],
)


def _prep_body(t0, k0, t1, k1, t2, k2, csum, a0, b0, a1, b1, a2, b2):
    csum[...] = k0[...] + k1[...] + k2[...]
    a0[...] = k0[...] * jnp.cos(t0[...])
    b0[...] = k0[...] * jnp.sin(t0[...])
    a1[...] = k1[...] * jnp.cos(t1[...])
    b1[...] = k1[...] * jnp.sin(t1[...])
    a2[...] = k2[...] * jnp.cos(t2[...])
    b2[...] = k2[...] * jnp.sin(t2[...])


def _finish_body(x, o):
    o[...] = jnp.sum(x[...], axis=0)


def kernel(pos, mapping, mapping_batch, atom_types,
           theta_0, k_0, theta_1, k_1, theta_2, k_2):
    ntab = N_TYPES ** 4
    shape2d = (ntab // 128, 128)
    tabs = [a.reshape(shape2d) for a in
            (theta_0, k_0, theta_1, k_1, theta_2, k_2)]
    coef = pl.pallas_call(
        _prep_body,
        out_shape=[jax.ShapeDtypeStruct(shape2d, _f32)] * 7,
    )(*tabs)

    nodes_f, ptab_f = _pack_kernel(
        pos.reshape(-1), atom_types, *[c.reshape(-1) for c in coef])
    nodes = nodes_f.reshape(N_NODES, ROW)
    ptab = ptab_f.reshape(ntab, ROW)

    part = _sc_kernel(nodes, mapping[0], mapping[1], mapping[2],
                      mapping[3], mapping_batch, ptab)

    eng = pl.pallas_call(
        _finish_body,
        out_shape=jax.ShapeDtypeStruct((ACC_SLOTS // 128, 128), _f32),
    )(part.reshape(NW, ACC_SLOTS // 128, 128))
    return eng.reshape(ACC_SLOTS)[:N_BATCH]


# speedup vs baseline: 264.2859x; 1.4180x over previous
import functools

import jax
import jax.numpy as jnp
from jax import lax
from jax.experimental import pallas as pl
from jax.experimental.pallas import tpu as pltpu
from jax.experimental.pallas import tpu_sc as plsc

N_NODES = 100000
N_DIH = 1600000
N_TYPES = 20
N_BATCH = 1000

NC = 2
NS = 16
NW = NC * NS
L = 16

CHUNK = 512
NSUB = CHUNK // 128
CHUNKS = 98
PER_TILE = CHUNKS * CHUNK
PADN = (NW - 1) * PER_TILE + (CHUNKS + 2) * CHUNK

ACC_SLOTS = 1024
ROW = 16

_f32 = jnp.float32
_i32 = jnp.int32


def _spl_f(v):
    return jnp.full((L,), v, _f32)


def _spl_i(v):
    return jnp.full((L,), v, _i32)


def _rsqrt16(x):
    xi = plsc.bitcast(x, _i32)
    yi = _spl_i(0x5F3759DF) - (xi >> 1)
    y = plsc.bitcast(yi, _f32)
    half_x = _spl_f(0.5) * x
    for _ in range(3):
        y = y * (_spl_f(1.5) - half_x * y * y)
    return y


def _sc_body(nodes, m0, m1, m2, m3, batp, ptab, out,
             map_v, bat_v, pos_v, pidx_v, cos_v, sin_v, par_v, acc_v, eng_v,
             gsem, msem, psem):
    mrows = (m0, m1, m2, m3)
    cid = lax.axis_index("c")
    sid = lax.axis_index("s")
    wid = sid * NC + cid
    tile_base = wid * PER_TILE

    lane = lax.iota(_i32, L)

    def _zero(i, carry):
        acc_v[pl.ds(i * L, L)] = _spl_f(0.0)
        return carry
    lax.fori_loop(0, (L * ACC_SLOTS) // L, _zero, 0)

    MSLOT = 4 * CHUNK
    PSLOT = 4 * CHUNK

    def fire_map(ci, slot):
        base = tile_base + ci * CHUNK
        for j in range(4):
            pltpu.async_copy(mrows[j].at[pl.ds(base, CHUNK)],
                             map_v.at[pl.ds(slot * MSLOT + j * CHUNK, CHUNK)],
                             msem)
        pltpu.async_copy(batp.at[pl.ds(base, CHUNK)],
                         bat_v.at[pl.ds(slot * CHUNK, CHUNK)], msem)

    def drain_map():
        for j in range(4):
            pltpu.make_async_copy(m0.at[pl.ds(0, CHUNK)],
                                  map_v.at[pl.ds(j * CHUNK, CHUNK)],
                                  msem).wait()
        pltpu.make_async_copy(batp.at[pl.ds(0, CHUNK)],
                              bat_v.at[pl.ds(0, CHUNK)], msem).wait()

    def fire_gathers(slot):
        for j in range(4):
            for kk in range(NSUB):
                idx = map_v.at[pl.ds(slot * MSLOT + j * CHUNK + kk * 128, 128)]
                dst = pos_v.at[pl.ds(slot * PSLOT + (j * NSUB + kk) * 128, 128)]
                pltpu.async_copy(nodes.at[idx], dst, gsem)

    def drain_gathers():
        for j in range(4):
            for kk in range(NSUB):
                dst = pos_v.at[pl.ds((j * NSUB + kk) * 128, 128)]
                pltpu.make_async_copy(nodes.at[pl.ds(0, 128)], dst, gsem).wait()

    def _pass_a(kk, pbase):
        def body(g8, carry):
            off = g8 * L
            row = lane + off

            def ld(j, c):
                return plsc.load_gather(
                    pos_v, [pbase + _spl_i((j * NSUB + kk) * 128) + row,
                            _spl_i(c)])

            p = [[ld(j, c) for c in range(3)] for j in range(4)]
            ti = [ld(j, 3).astype(_i32) for j in range(4)]

            b0 = [p[0][c] - p[1][c] for c in range(3)]
            b1 = [p[2][c] - p[1][c] for c in range(3)]
            b2 = [p[3][c] - p[2][c] for c in range(3)]
            s = b1[0] * b1[0] + b1[1] * b1[1] + b1[2] * b1[2]
            d01 = b0[0] * b1[0] + b0[1] * b1[1] + b0[2] * b1[2]
            d21 = b2[0] * b1[0] + b2[1] * b1[1] + b2[2] * b1[2]
            d02 = b0[0] * b2[0] + b0[1] * b2[1] + b0[2] * b2[2]
            crx = b0[1] * b2[2] - b0[2] * b2[1]
            cry = b0[2] * b2[0] - b0[0] * b2[2]
            crz = b0[0] * b2[1] - b0[1] * b2[0]
            tt = b1[0] * crx + b1[1] * cry + b1[2] * crz

            zero = _spl_f(0.0)
            one = _spl_f(1.0)
            szero = s == zero
            rs = _rsqrt16(jnp.where(szero, one, s))
            x = s * d02 - d01 * d21
            y = s * rs * tt
            x = jnp.where(szero, d02, x)
            y = jnp.where(szero, zero, y)
            r2 = x * x + y * y
            r2z = r2 == zero
            inv = _rsqrt16(jnp.where(r2z, one, r2))
            cosv = jnp.where(r2z, one, x * inv)
            sinv = jnp.where(r2z, zero, y * inv)

            goff = kk * 128 + off
            cos_v[pl.ds(goff, L)] = cosv
            sin_v[pl.ds(goff, L)] = sinv

            twenty = _spl_i(N_TYPES)
            pidx = ((ti[0] * twenty + ti[1]) * twenty + ti[2]) * twenty + ti[3]
            pidx_v[kk, pl.ds(off, L)] = pidx
            return carry
        return body

    def _pass_b(kk, bbase):
        def body(g8, carry):
            off = g8 * L
            goff = kk * 128 + off
            row = lane + off + kk * 128
            pr = [plsc.load_gather(par_v, [row, _spl_i(c)]) for c in range(7)]
            cosv = cos_v[pl.ds(goff, L)]
            sinv = sin_v[pl.ds(goff, L)]
            bid = bat_v[pl.ds(bbase + goff, L)]
            one = _spl_f(1.0)
            two = _spl_f(2.0)
            c2 = two * cosv * cosv - one
            s2 = two * sinv * cosv
            dd = two * c2
            c3 = cosv * (dd - one)
            s3 = sinv * (dd + one)
            v = pr[0] - (pr[1] * cosv + pr[2] * sinv + pr[3] * c2 +
                         pr[4] * s2 + pr[5] * c3 + pr[6] * s3)
            addr = lane * _spl_i(ACC_SLOTS) + bid
            plsc.addupdate_scatter(acc_v, [addr], v)
            return carry
        return body

    for j in range(4):
        pltpu.sync_copy(mrows[j].at[pl.ds(tile_base, CHUNK)],
                        map_v.at[pl.ds(j * CHUNK, CHUNK)])
    pltpu.sync_copy(batp.at[pl.ds(tile_base, CHUNK)],
                    bat_v.at[pl.ds(0, CHUNK)])
    fire_gathers(0)
    fire_map(1, 1)

    def loop(i, carry):
        p = i & 1
        q = 1 - p
        drain_gathers()
        drain_map()
        fire_gathers(q)
        pbase = jnp.broadcast_to(p * PSLOT, (L,)).astype(_i32)
        for kk in range(NSUB):
            lax.fori_loop(0, 128 // L, _pass_a(kk, pbase), 0)
            pltpu.async_copy(ptab.at[pidx_v.at[kk]],
                             par_v.at[pl.ds(kk * 128, 128)], psem)
        for kk in range(NSUB):
            pltpu.make_async_copy(ptab.at[pl.ds(0, 128)],
                                  par_v.at[pl.ds(kk * 128, 128)], psem).wait()
        bbase = p * CHUNK
        for kk in range(NSUB):
            lax.fori_loop(0, 128 // L, _pass_b(kk, bbase), 0)
        fire_map(i + 2, p)
        return carry
    lax.fori_loop(0, CHUNKS, loop, 0)
    drain_gathers()
    drain_map()

    def _fold(j, carry):
        col = j * L
        v = acc_v[pl.ds(col, L)]
        for r in range(1, L):
            v = v + acc_v[pl.ds(r * ACC_SLOTS + col, L)]
        eng_v[pl.ds(col, L)] = v
        return carry
    lax.fori_loop(0, ACC_SLOTS // L, _fold, 0)

    pltpu.sync_copy(eng_v, out.at[pl.ds(wid * ACC_SLOTS, ACC_SLOTS)])


_sc_kernel = pl.kernel(
    _sc_body,
    out_type=jax.ShapeDtypeStruct((NW * ACC_SLOTS,), _f32),
    mesh=plsc.VectorSubcoreMesh(core_axis_name="c", subcore_axis_name="s"),
    compiler_params=pltpu.CompilerParams(
        needs_layout_passes=False, use_tc_tiling_on_sc=False),
    scratch_types=[
        pltpu.VMEM((2 * 4 * CHUNK,), _i32),
        pltpu.VMEM((2 * CHUNK,), _i32),
        pltpu.VMEM((2 * 4 * CHUNK, ROW), _f32),
        pltpu.VMEM((NSUB, 128), _i32),
        pltpu.VMEM((CHUNK,), _f32),
        pltpu.VMEM((CHUNK,), _f32),
        pltpu.VMEM((CHUNK, ROW), _f32),
        pltpu.VMEM((L * ACC_SLOTS,), _f32),
        pltpu.VMEM((ACC_SLOTS,), _f32),
        pltpu.SemaphoreType.DMA,
        pltpu.SemaphoreType.DMA,
        pltpu.SemaphoreType.DMA,
    ],
)


PK_N_CNT = 3136
PK_N_STRIDE = 3128
PK_P_CNT = 5008
PK_P_STRIDE = 5000
PK_P_HALF = (2512, 2496)


def _pack_body(posf, typ, c0, c1, c2, c3, c4, c5, c6,
               nodes_out, ptab_out,
               posb, typb, packb, cb0, cb1, cb2, cb3, cb4, cb5, cb6, packp,
               sem):
    cid = lax.axis_index("c")
    sid = lax.axis_index("s")
    wid = sid * NC + cid
    lane = lax.iota(_i32, L)

    nbase = jnp.minimum(wid * PK_N_STRIDE, N_NODES - PK_N_CNT)
    pltpu.sync_copy(posf.at[pl.ds(nbase * 3, PK_N_CNT * 3)], posb)
    pltpu.sync_copy(typ.at[pl.ds(nbase, PK_N_CNT)], typb)
    for h in range(2):
        hoff = h * (PK_N_CNT // 2)

        def ngrp(gl, carry):
            lh = lane + gl * L
            src = (lh + hoff) * _spl_i(3)
            dst = lh * _spl_i(ROW)
            for c in range(3):
                v = plsc.load_gather(posb, [src + _spl_i(c)])
                plsc.store_scatter(packb, [dst + _spl_i(c)], v)
            tv = typb[pl.ds(hoff + gl * L, L)].astype(_f32)
            plsc.store_scatter(packb, [dst + _spl_i(3)], tv)
            return carry
        lax.fori_loop(0, PK_N_CNT // 2 // L, ngrp, 0)
        pltpu.sync_copy(
            packb, nodes_out.at[pl.ds((nbase + hoff) * ROW, PK_N_CNT // 2 * ROW)])

    pbase = jnp.minimum(wid * PK_P_STRIDE, N_TYPES ** 4 - PK_P_CNT)
    cbs = [cb0, cb1, cb2, cb3, cb4, cb5, cb6]
    planes = [c0, c1, c2, c3, c4, c5, c6]
    for h in range(2):
        hoff = h * PK_P_HALF[0]
        cnt = PK_P_HALF[h]
        for cb, pf in zip(cbs, planes):
            pltpu.sync_copy(pf.at[pl.ds(pbase + hoff, cnt)],
                            cb.at[pl.ds(0, cnt)])

        def pgrp(gl, carry):
            lh = lane + gl * L
            dst = lh * _spl_i(ROW)
            for c in range(7):
                v = cbs[c][pl.ds(gl * L, L)]
                plsc.store_scatter(packp, [dst + _spl_i(c)], v)
            return carry
        lax.fori_loop(0, cnt // L, pgrp, 0)
        pltpu.sync_copy(packp.at[pl.ds(0, cnt * ROW)],
                        ptab_out.at[pl.ds((pbase + hoff) * ROW, cnt * ROW)])


_pack_kernel = pl.kernel(
    _pack_body,
    out_type=(jax.ShapeDtypeStruct((N_NODES * ROW,), _f32),
              jax.ShapeDtypeStruct((N_TYPES ** 4 * ROW,), _f32)),
    mesh=plsc.VectorSubcoreMesh(core_axis_name="c", subcore_axis_name="s"),
    compiler_params=pltpu.CompilerParams(
        needs_layout_passes=False, use_tc_tiling_on_sc=False),
    scratch_types=[
        pltpu.VMEM((PK_N_CNT * 3,), _f32),
        pltpu.VMEM((PK_N_CNT,), _i32),
        pltpu.VMEM((PK_N_CNT // 2 * ROW,), _f32),
    ] + [pltpu.VMEM((PK_P_HALF[0],), _f32)] * 7
    + [
        pltpu.VMEM((PK_P_HALF[0] * ROW,), _f32),
        pltpu.SemaphoreType.DMA,
    ],
)


def _prep_body(t0, k0, t1, k1, t2, k2, csum, a0, b0, a1, b1, a2, b2):
    csum[...] = k0[...] + k1[...] + k2[...]
    a0[...] = k0[...] * jnp.cos(t0[...])
    b0[...] = k0[...] * jnp.sin(t0[...])
    a1[...] = k1[...] * jnp.cos(t1[...])
    b1[...] = k1[...] * jnp.sin(t1[...])
    a2[...] = k2[...] * jnp.cos(t2[...])
    b2[...] = k2[...] * jnp.sin(t2[...])


def _finish_body(x, o):
    o[...] = jnp.sum(x[...], axis=0)


def kernel(pos, mapping, mapping_batch, atom_types,
           theta_0, k_0, theta_1, k_1, theta_2, k_2):
    ntab = N_TYPES ** 4
    shape2d = (ntab // 128, 128)
    tabs = [a.reshape(shape2d) for a in
            (theta_0, k_0, theta_1, k_1, theta_2, k_2)]
    coef = pl.pallas_call(
        _prep_body,
        out_shape=[jax.ShapeDtypeStruct(shape2d, _f32)] * 7,
    )(*tabs)

    nodes_f, ptab_f = _pack_kernel(
        pos.reshape(-1), atom_types, *[c.reshape(-1) for c in coef])
    nodes = nodes_f.reshape(N_NODES, ROW)
    ptab = ptab_f.reshape(ntab, ROW)

    mpad = [jnp.pad(mapping[j], (0, PADN - N_DIH)) for j in range(4)]
    bpad = jnp.pad(mapping_batch, (0, PADN - N_DIH),
                   constant_values=N_BATCH)
    part = _sc_kernel(nodes, mpad[0], mpad[1], mpad[2], mpad[3], bpad, ptab)

    eng = pl.pallas_call(
        _finish_body,
        out_shape=jax.ShapeDtypeStruct((ACC_SLOTS // 128, 128), _f32),
    )(part.reshape(NW, ACC_SLOTS // 128, 128))
    return eng.reshape(ACC_SLOTS)[:N_BATCH]


# speedup vs baseline: 264.3972x; 1.0004x over previous
import functools

import jax
import jax.numpy as jnp
from jax import lax
from jax.experimental import pallas as pl
from jax.experimental.pallas import tpu as pltpu
from jax.experimental.pallas import tpu_sc as plsc

N_NODES = 100000
N_DIH = 1600000
N_TYPES = 20
N_BATCH = 1000

NC = 2
NS = 16
NW = NC * NS
L = 16

CHUNK = 512
NSUB = CHUNK // 128
CHUNKS = 98
PER_TILE = CHUNKS * CHUNK
PADN = (NW - 1) * PER_TILE + (CHUNKS + 2) * CHUNK

ACC_SLOTS = 1024
ROW = 16

_f32 = jnp.float32
_i32 = jnp.int32


def _spl_f(v):
    return jnp.full((L,), v, _f32)


def _spl_i(v):
    return jnp.full((L,), v, _i32)


def _rsqrt16(x):
    xi = plsc.bitcast(x, _i32)
    yi = _spl_i(0x5F3759DF) - (xi >> 1)
    y = plsc.bitcast(yi, _f32)
    half_x = _spl_f(0.5) * x
    for _ in range(3):
        y = y * (_spl_f(1.5) - half_x * y * y)
    return y


def _sc_body(nodes, m0, m1, m2, m3, batp, ptab, out,
             map_v, bat_v, pos_v, pidx_v, cos_v, sin_v, par_v, acc_v, eng_v,
             gsem, msem, psem):
    mrows = (m0, m1, m2, m3)
    cid = lax.axis_index("c")
    sid = lax.axis_index("s")
    wid = sid * NC + cid
    tile_base = wid * PER_TILE

    lane = lax.iota(_i32, L)

    def _zero(i, carry):
        acc_v[pl.ds(i * L, L)] = _spl_f(0.0)
        return carry
    lax.fori_loop(0, (L * ACC_SLOTS) // L, _zero, 0)

    MSLOT = 4 * CHUNK
    PSLOT = 4 * CHUNK

    def fire_map(ci, slot):
        base = tile_base + ci * CHUNK
        for j in range(4):
            pltpu.async_copy(mrows[j].at[pl.ds(base, CHUNK)],
                             map_v.at[pl.ds(slot * MSLOT + j * CHUNK, CHUNK)],
                             msem)
        pltpu.async_copy(batp.at[pl.ds(base, CHUNK)],
                         bat_v.at[pl.ds(slot * CHUNK, CHUNK)], msem)

    def drain_map():
        for j in range(4):
            pltpu.make_async_copy(m0.at[pl.ds(0, CHUNK)],
                                  map_v.at[pl.ds(j * CHUNK, CHUNK)],
                                  msem).wait()
        pltpu.make_async_copy(batp.at[pl.ds(0, CHUNK)],
                              bat_v.at[pl.ds(0, CHUNK)], msem).wait()

    def fire_gathers(slot):
        for j in range(4):
            for kk in range(NSUB):
                idx = map_v.at[pl.ds(slot * MSLOT + j * CHUNK + kk * 128, 128)]
                dst = pos_v.at[pl.ds(slot * PSLOT + (j * NSUB + kk) * 128, 128)]
                pltpu.async_copy(nodes.at[idx], dst, gsem)

    def drain_gathers():
        for j in range(4):
            for kk in range(NSUB):
                dst = pos_v.at[pl.ds((j * NSUB + kk) * 128, 128)]
                pltpu.make_async_copy(nodes.at[pl.ds(0, 128)], dst, gsem).wait()

    def _pass_a(kk, pbase):
        def body(g8, carry):
            off = g8 * L
            row = lane + off

            def ld(j, c):
                return plsc.load_gather(
                    pos_v, [pbase + _spl_i((j * NSUB + kk) * 128) + row,
                            _spl_i(c)])

            p = [[ld(j, c) for c in range(3)] for j in range(4)]
            ti = [ld(j, 3).astype(_i32) for j in range(4)]

            b0 = [p[0][c] - p[1][c] for c in range(3)]
            b1 = [p[2][c] - p[1][c] for c in range(3)]
            b2 = [p[3][c] - p[2][c] for c in range(3)]
            s = b1[0] * b1[0] + b1[1] * b1[1] + b1[2] * b1[2]
            d01 = b0[0] * b1[0] + b0[1] * b1[1] + b0[2] * b1[2]
            d21 = b2[0] * b1[0] + b2[1] * b1[1] + b2[2] * b1[2]
            d02 = b0[0] * b2[0] + b0[1] * b2[1] + b0[2] * b2[2]
            crx = b0[1] * b2[2] - b0[2] * b2[1]
            cry = b0[2] * b2[0] - b0[0] * b2[2]
            crz = b0[0] * b2[1] - b0[1] * b2[0]
            tt = b1[0] * crx + b1[1] * cry + b1[2] * crz

            zero = _spl_f(0.0)
            one = _spl_f(1.0)
            szero = s == zero
            rs = _rsqrt16(jnp.where(szero, one, s))
            x = s * d02 - d01 * d21
            y = s * rs * tt
            x = jnp.where(szero, d02, x)
            y = jnp.where(szero, zero, y)
            r2 = x * x + y * y
            r2z = r2 == zero
            inv = _rsqrt16(jnp.where(r2z, one, r2))
            cosv = jnp.where(r2z, one, x * inv)
            sinv = jnp.where(r2z, zero, y * inv)

            goff = kk * 128 + off
            cos_v[pl.ds(goff, L)] = cosv
            sin_v[pl.ds(goff, L)] = sinv

            twenty = _spl_i(N_TYPES)
            pidx = ((ti[0] * twenty + ti[1]) * twenty + ti[2]) * twenty + ti[3]
            pidx_v[kk, pl.ds(off, L)] = pidx
            return carry
        return body

    def _pass_b(kk, bbase):
        def body(g8, carry):
            off = g8 * L
            goff = kk * 128 + off
            row = lane + off + kk * 128
            pr = [plsc.load_gather(par_v, [row, _spl_i(c)]) for c in range(7)]
            cosv = cos_v[pl.ds(goff, L)]
            sinv = sin_v[pl.ds(goff, L)]
            bid = bat_v[pl.ds(bbase + goff, L)]
            one = _spl_f(1.0)
            two = _spl_f(2.0)
            c2 = two * cosv * cosv - one
            s2 = two * sinv * cosv
            dd = two * c2
            c3 = cosv * (dd - one)
            s3 = sinv * (dd + one)
            v = pr[0] - (pr[1] * cosv + pr[2] * sinv + pr[3] * c2 +
                         pr[4] * s2 + pr[5] * c3 + pr[6] * s3)
            addr = lane * _spl_i(ACC_SLOTS) + bid
            plsc.addupdate_scatter(acc_v, [addr], v)
            return carry
        return body

    for j in range(4):
        pltpu.sync_copy(mrows[j].at[pl.ds(tile_base, CHUNK)],
                        map_v.at[pl.ds(j * CHUNK, CHUNK)])
    pltpu.sync_copy(batp.at[pl.ds(tile_base, CHUNK)],
                    bat_v.at[pl.ds(0, CHUNK)])
    fire_gathers(0)
    fire_map(1, 1)

    def loop(i, carry):
        p = i & 1
        q = 1 - p
        drain_gathers()
        drain_map()
        fire_gathers(q)
        pbase = jnp.broadcast_to(p * PSLOT, (L,)).astype(_i32)
        for kk in range(NSUB):
            lax.fori_loop(0, 128 // L, _pass_a(kk, pbase), 0)
            pltpu.async_copy(ptab.at[pidx_v.at[kk]],
                             par_v.at[pl.ds(kk * 128, 128)], psem)
        for kk in range(NSUB):
            pltpu.make_async_copy(ptab.at[pl.ds(0, 128)],
                                  par_v.at[pl.ds(kk * 128, 128)], psem).wait()
        bbase = p * CHUNK
        for kk in range(NSUB):
            lax.fori_loop(0, 128 // L, _pass_b(kk, bbase), 0)
        fire_map(i + 2, p)
        return carry
    lax.fori_loop(0, CHUNKS, loop, 0)
    drain_gathers()
    drain_map()

    def _fold(j, carry):
        col = j * L
        v = acc_v[pl.ds(col, L)]
        for r in range(1, L):
            v = v + acc_v[pl.ds(r * ACC_SLOTS + col, L)]
        eng_v[pl.ds(col, L)] = v
        return carry
    lax.fori_loop(0, ACC_SLOTS // L, _fold, 0)

    pltpu.sync_copy(eng_v, out.at[pl.ds(wid * ACC_SLOTS, ACC_SLOTS)])


_sc_kernel = pl.kernel(
    _sc_body,
    out_type=jax.ShapeDtypeStruct((NW * ACC_SLOTS,), _f32),
    mesh=plsc.VectorSubcoreMesh(core_axis_name="c", subcore_axis_name="s"),
    compiler_params=pltpu.CompilerParams(
        needs_layout_passes=False, use_tc_tiling_on_sc=False),
    scratch_types=[
        pltpu.VMEM((2 * 4 * CHUNK,), _i32),
        pltpu.VMEM((2 * CHUNK,), _i32),
        pltpu.VMEM((2 * 4 * CHUNK, ROW), _f32),
        pltpu.VMEM((NSUB, 128), _i32),
        pltpu.VMEM((CHUNK,), _f32),
        pltpu.VMEM((CHUNK,), _f32),
        pltpu.VMEM((CHUNK, ROW), _f32),
        pltpu.VMEM((L * ACC_SLOTS,), _f32),
        pltpu.VMEM((ACC_SLOTS,), _f32),
        pltpu.SemaphoreType.DMA,
        pltpu.SemaphoreType.DMA,
        pltpu.SemaphoreType.DMA,
    ],
)


PK_N_CNT = 3136
PK_N_STRIDE = 3128
PK_P_CNT = 5008
PK_P_STRIDE = 5000
PK_P_HALF = (2512, 2496)


def _pack_body(posf, typ, c0, c1, c2, c3, c4, c5, c6,
               nodes_out, ptab_out,
               posb, typb, packb, cb0, cb1, cb2, cb3, cb4, cb5, cb6, packp,
               sem):
    cid = lax.axis_index("c")
    sid = lax.axis_index("s")
    wid = sid * NC + cid
    lane = lax.iota(_i32, L)

    nbase = jnp.minimum(wid * PK_N_STRIDE, N_NODES - PK_N_CNT)
    pltpu.sync_copy(posf.at[pl.ds(nbase * 3, PK_N_CNT * 3)], posb)
    pltpu.sync_copy(typ.at[pl.ds(nbase, PK_N_CNT)], typb)
    for h in range(2):
        hoff = h * (PK_N_CNT // 2)

        def ngrp(gl, carry):
            lh = lane + gl * L
            src = (lh + hoff) * _spl_i(3)
            for c in range(3):
                v = plsc.load_gather(posb, [src + _spl_i(c)])
                plsc.store_scatter(packb, [lh, _spl_i(c)], v)
            tv = typb[pl.ds(hoff + gl * L, L)].astype(_f32)
            plsc.store_scatter(packb, [lh, _spl_i(3)], tv)
            return carry
        lax.fori_loop(0, PK_N_CNT // 2 // L, ngrp, 0)
        pltpu.sync_copy(
            packb, nodes_out.at[pl.ds(nbase + hoff, PK_N_CNT // 2)])

    pbase = jnp.minimum(wid * PK_P_STRIDE, N_TYPES ** 4 - PK_P_CNT)
    cbs = [cb0, cb1, cb2, cb3, cb4, cb5, cb6]
    planes = [c0, c1, c2, c3, c4, c5, c6]
    for h in range(2):
        hoff = h * PK_P_HALF[0]
        cnt = PK_P_HALF[h]
        for cb, pf in zip(cbs, planes):
            pltpu.sync_copy(pf.at[pl.ds(pbase + hoff, cnt)],
                            cb.at[pl.ds(0, cnt)])

        def pgrp(gl, carry):
            lh = lane + gl * L
            for c in range(7):
                v = cbs[c][pl.ds(gl * L, L)]
                plsc.store_scatter(packp, [lh, _spl_i(c)], v)
            return carry
        lax.fori_loop(0, cnt // L, pgrp, 0)
        pltpu.sync_copy(packp.at[pl.ds(0, cnt)],
                        ptab_out.at[pl.ds(pbase + hoff, cnt)])


_pack_kernel = pl.kernel(
    _pack_body,
    out_type=(jax.ShapeDtypeStruct((N_NODES, ROW), _f32),
              jax.ShapeDtypeStruct((N_TYPES ** 4, ROW), _f32)),
    mesh=plsc.VectorSubcoreMesh(core_axis_name="c", subcore_axis_name="s"),
    compiler_params=pltpu.CompilerParams(
        needs_layout_passes=False, use_tc_tiling_on_sc=False),
    scratch_types=[
        pltpu.VMEM((PK_N_CNT * 3,), _f32),
        pltpu.VMEM((PK_N_CNT,), _i32),
        pltpu.VMEM((PK_N_CNT // 2, ROW), _f32),
    ] + [pltpu.VMEM((PK_P_HALF[0],), _f32)] * 7
    + [
        pltpu.VMEM((PK_P_HALF[0], ROW), _f32),
        pltpu.SemaphoreType.DMA,
    ],
)


def _prep_body(t0, k0, t1, k1, t2, k2, csum, a0, b0, a1, b1, a2, b2):
    csum[...] = k0[...] + k1[...] + k2[...]
    a0[...] = k0[...] * jnp.cos(t0[...])
    b0[...] = k0[...] * jnp.sin(t0[...])
    a1[...] = k1[...] * jnp.cos(t1[...])
    b1[...] = k1[...] * jnp.sin(t1[...])
    a2[...] = k2[...] * jnp.cos(t2[...])
    b2[...] = k2[...] * jnp.sin(t2[...])


def _finish_body(x, o):
    o[...] = jnp.sum(x[...], axis=0)


def kernel(pos, mapping, mapping_batch, atom_types,
           theta_0, k_0, theta_1, k_1, theta_2, k_2):
    ntab = N_TYPES ** 4
    shape2d = (ntab // 128, 128)
    tabs = [a.reshape(shape2d) for a in
            (theta_0, k_0, theta_1, k_1, theta_2, k_2)]
    coef = pl.pallas_call(
        _prep_body,
        out_shape=[jax.ShapeDtypeStruct(shape2d, _f32)] * 7,
    )(*tabs)

    nodes, ptab = _pack_kernel(
        pos.reshape(-1), atom_types, *[c.reshape(-1) for c in coef])

    mpad = [jnp.pad(mapping[j], (0, PADN - N_DIH)) for j in range(4)]
    bpad = jnp.pad(mapping_batch, (0, PADN - N_DIH),
                   constant_values=N_BATCH)
    part = _sc_kernel(nodes, mpad[0], mpad[1], mpad[2], mpad[3], bpad, ptab)

    eng = pl.pallas_call(
        _finish_body,
        out_shape=jax.ShapeDtypeStruct((ACC_SLOTS // 128, 128), _f32),
    )(part.reshape(NW, ACC_SLOTS // 128, 128))
    return eng.reshape(ACC_SLOTS)[:N_BATCH]


# speedup vs baseline: 281.2582x; 1.0638x over previous
import functools

import jax
import jax.numpy as jnp
from jax import lax
from jax.experimental import pallas as pl
from jax.experimental.pallas import tpu as pltpu
from jax.experimental.pallas import tpu_sc as plsc

N_NODES = 100000
N_DIH = 1600000
N_TYPES = 20
N_BATCH = 1000

NC = 2
NS = 16
NW = NC * NS
L = 16

CHUNK = 512
NSUB = CHUNK // 128
CHUNKS = 98
PER_TILE = CHUNKS * CHUNK
PADN = (NW - 1) * PER_TILE + (CHUNKS + 2) * CHUNK

ACC_SLOTS = 1024
ROW = 16

_f32 = jnp.float32
_i32 = jnp.int32


def _spl_f(v):
    return jnp.full((L,), v, _f32)


def _spl_i(v):
    return jnp.full((L,), v, _i32)


def _rsqrt16(x):
    xi = plsc.bitcast(x, _i32)
    yi = _spl_i(0x5F3759DF) - (xi >> 1)
    y = plsc.bitcast(yi, _f32)
    half_x = _spl_f(0.5) * x
    for _ in range(3):
        y = y * (_spl_f(1.5) - half_x * y * y)
    return y


def _sc_body(nodes, m0, m1, m2, m3, batp, ptab, out,
             map_v, bat_v, pos_v, pidx_v, cos_v, sin_v, par_v, acc_v, eng_v,
             gsem, msem, psem):
    mrows = (m0, m1, m2, m3)
    cid = lax.axis_index("c")
    sid = lax.axis_index("s")
    wid = sid * NC + cid
    tile_base = wid * PER_TILE

    lane = lax.iota(_i32, L)

    def _zero(i, carry):
        acc_v[pl.ds(i * L, L)] = _spl_f(0.0)
        return carry
    lax.fori_loop(0, (L * ACC_SLOTS) // L, _zero, 0)

    MSLOT = 4 * CHUNK
    PSLOT = 4 * CHUNK

    def fire_map(ci, slot):
        base = tile_base + ci * CHUNK
        for j in range(4):
            pltpu.async_copy(mrows[j].at[pl.ds(base, CHUNK)],
                             map_v.at[pl.ds(slot * MSLOT + j * CHUNK, CHUNK)],
                             msem)
        pltpu.async_copy(batp.at[pl.ds(base, CHUNK)],
                         bat_v.at[pl.ds(slot * CHUNK, CHUNK)], msem)

    def drain_map():
        for j in range(4):
            pltpu.make_async_copy(m0.at[pl.ds(0, CHUNK)],
                                  map_v.at[pl.ds(j * CHUNK, CHUNK)],
                                  msem).wait()
        pltpu.make_async_copy(batp.at[pl.ds(0, CHUNK)],
                              bat_v.at[pl.ds(0, CHUNK)], msem).wait()

    def fire_gathers(slot):
        for j in range(4):
            for kk in range(NSUB):
                idx = map_v.at[pl.ds(slot * MSLOT + j * CHUNK + kk * 128, 128)]
                dst = pos_v.at[pl.ds(slot * PSLOT + (j * NSUB + kk) * 128, 128)]
                pltpu.async_copy(nodes.at[idx], dst, gsem)

    def drain_gathers():
        for j in range(4):
            for kk in range(NSUB):
                dst = pos_v.at[pl.ds((j * NSUB + kk) * 128, 128)]
                pltpu.make_async_copy(nodes.at[pl.ds(0, 128)], dst, gsem).wait()

    def _pass_a(kk, pbase):
        def body(g8, carry):
            off = g8 * L
            row = lane + off

            def ld(j, c):
                return plsc.load_gather(
                    pos_v, [pbase + _spl_i((j * NSUB + kk) * 128) + row,
                            _spl_i(c)])

            p = [[ld(j, c) for c in range(3)] for j in range(4)]
            ti = [ld(j, 3).astype(_i32) for j in range(4)]

            b0 = [p[0][c] - p[1][c] for c in range(3)]
            b1 = [p[2][c] - p[1][c] for c in range(3)]
            b2 = [p[3][c] - p[2][c] for c in range(3)]
            s = b1[0] * b1[0] + b1[1] * b1[1] + b1[2] * b1[2]
            d01 = b0[0] * b1[0] + b0[1] * b1[1] + b0[2] * b1[2]
            d21 = b2[0] * b1[0] + b2[1] * b1[1] + b2[2] * b1[2]
            d02 = b0[0] * b2[0] + b0[1] * b2[1] + b0[2] * b2[2]
            crx = b0[1] * b2[2] - b0[2] * b2[1]
            cry = b0[2] * b2[0] - b0[0] * b2[2]
            crz = b0[0] * b2[1] - b0[1] * b2[0]
            tt = b1[0] * crx + b1[1] * cry + b1[2] * crz

            zero = _spl_f(0.0)
            one = _spl_f(1.0)
            szero = s == zero
            rs = _rsqrt16(jnp.where(szero, one, s))
            x = s * d02 - d01 * d21
            y = s * rs * tt
            x = jnp.where(szero, d02, x)
            y = jnp.where(szero, zero, y)
            r2 = x * x + y * y
            r2z = r2 == zero
            inv = _rsqrt16(jnp.where(r2z, one, r2))
            cosv = jnp.where(r2z, one, x * inv)
            sinv = jnp.where(r2z, zero, y * inv)

            goff = kk * 128 + off
            cos_v[pl.ds(goff, L)] = cosv
            sin_v[pl.ds(goff, L)] = sinv

            twenty = _spl_i(N_TYPES)
            pidx = ((ti[0] * twenty + ti[1]) * twenty + ti[2]) * twenty + ti[3]
            pidx_v[kk, pl.ds(off, L)] = pidx
            return carry
        return body

    def _pass_b(kk, bbase):
        def body(g8, carry):
            off = g8 * L
            goff = kk * 128 + off
            row = lane + off + kk * 128
            pr = [plsc.load_gather(par_v, [row, _spl_i(c)]) for c in range(7)]
            cosv = cos_v[pl.ds(goff, L)]
            sinv = sin_v[pl.ds(goff, L)]
            bid = bat_v[pl.ds(bbase + goff, L)]
            one = _spl_f(1.0)
            two = _spl_f(2.0)
            c2 = two * cosv * cosv - one
            s2 = two * sinv * cosv
            dd = two * c2
            c3 = cosv * (dd - one)
            s3 = sinv * (dd + one)
            v = pr[0] - (pr[1] * cosv + pr[2] * sinv + pr[3] * c2 +
                         pr[4] * s2 + pr[5] * c3 + pr[6] * s3)
            addr = lane * _spl_i(ACC_SLOTS) + bid
            plsc.addupdate_scatter(acc_v, [addr], v)
            return carry
        return body

    for j in range(4):
        pltpu.sync_copy(mrows[j].at[pl.ds(tile_base, CHUNK)],
                        map_v.at[pl.ds(j * CHUNK, CHUNK)])
    pltpu.sync_copy(batp.at[pl.ds(tile_base, CHUNK)],
                    bat_v.at[pl.ds(0, CHUNK)])
    fire_gathers(0)
    fire_map(1, 1)

    def loop(i, carry):
        p = i & 1
        q = 1 - p
        drain_gathers()
        drain_map()
        fire_gathers(q)
        pbase = jnp.broadcast_to(p * PSLOT, (L,)).astype(_i32)
        for kk in range(NSUB):
            lax.fori_loop(0, 128 // L, _pass_a(kk, pbase), 0)
            pltpu.async_copy(ptab.at[pidx_v.at[kk]],
                             par_v.at[pl.ds(kk * 128, 128)], psem)
        bbase = p * CHUNK
        for kk in range(NSUB):
            pltpu.make_async_copy(ptab.at[pl.ds(0, 128)],
                                  par_v.at[pl.ds(kk * 128, 128)], psem).wait()
            lax.fori_loop(0, 128 // L, _pass_b(kk, bbase), 0)
        fire_map(i + 2, p)
        return carry
    lax.fori_loop(0, CHUNKS, loop, 0)
    drain_gathers()
    drain_map()

    def _fold(j, carry):
        col = j * L
        v = acc_v[pl.ds(col, L)]
        for r in range(1, L):
            v = v + acc_v[pl.ds(r * ACC_SLOTS + col, L)]
        eng_v[pl.ds(col, L)] = v
        return carry
    lax.fori_loop(0, ACC_SLOTS // L, _fold, 0)

    pltpu.sync_copy(eng_v, out.at[pl.ds(wid * ACC_SLOTS, ACC_SLOTS)])


_sc_kernel = pl.kernel(
    _sc_body,
    out_type=jax.ShapeDtypeStruct((NW * ACC_SLOTS,), _f32),
    mesh=plsc.VectorSubcoreMesh(core_axis_name="c", subcore_axis_name="s"),
    compiler_params=pltpu.CompilerParams(
        needs_layout_passes=False, use_tc_tiling_on_sc=False),
    scratch_types=[
        pltpu.VMEM((2 * 4 * CHUNK,), _i32),
        pltpu.VMEM((2 * CHUNK,), _i32),
        pltpu.VMEM((2 * 4 * CHUNK, ROW), _f32),
        pltpu.VMEM((NSUB, 128), _i32),
        pltpu.VMEM((CHUNK,), _f32),
        pltpu.VMEM((CHUNK,), _f32),
        pltpu.VMEM((CHUNK, ROW), _f32),
        pltpu.VMEM((L * ACC_SLOTS,), _f32),
        pltpu.VMEM((ACC_SLOTS,), _f32),
        pltpu.SemaphoreType.DMA,
        pltpu.SemaphoreType.DMA,
        pltpu.SemaphoreType.DMA,
    ],
)


PK_N_CNT = 3136
PK_N_STRIDE = 3128
PK_P_CNT = 5008
PK_P_STRIDE = 5000
PK_P_HALF = (2512, 2496)


def _pack_body(posf, typ, c0, c1, c2, c3, c4, c5, c6,
               nodes_out, ptab_out,
               posb, typb, packb, cb0, cb1, cb2, cb3, cb4, cb5, cb6, packp,
               sem):
    cid = lax.axis_index("c")
    sid = lax.axis_index("s")
    wid = sid * NC + cid
    lane = lax.iota(_i32, L)

    nbase = jnp.minimum(wid * PK_N_STRIDE, N_NODES - PK_N_CNT)
    pltpu.sync_copy(posf.at[pl.ds(nbase * 3, PK_N_CNT * 3)], posb)
    pltpu.sync_copy(typ.at[pl.ds(nbase, PK_N_CNT)], typb)
    for h in range(2):
        hoff = h * (PK_N_CNT // 2)

        def ngrp(gl, carry):
            lh = lane + gl * L
            src = (lh + hoff) * _spl_i(3)
            for c in range(3):
                v = plsc.load_gather(posb, [src + _spl_i(c)])
                plsc.store_scatter(packb, [lh, _spl_i(c)], v)
            tv = typb[pl.ds(hoff + gl * L, L)].astype(_f32)
            plsc.store_scatter(packb, [lh, _spl_i(3)], tv)
            return carry
        lax.fori_loop(0, PK_N_CNT // 2 // L, ngrp, 0)
        pltpu.sync_copy(
            packb, nodes_out.at[pl.ds(nbase + hoff, PK_N_CNT // 2)])

    pbase = jnp.minimum(wid * PK_P_STRIDE, N_TYPES ** 4 - PK_P_CNT)
    cbs = [cb0, cb1, cb2, cb3, cb4, cb5, cb6]
    planes = [c0, c1, c2, c3, c4, c5, c6]
    for h in range(2):
        hoff = h * PK_P_HALF[0]
        cnt = PK_P_HALF[h]
        for cb, pf in zip(cbs, planes):
            pltpu.sync_copy(pf.at[pl.ds(pbase + hoff, cnt)],
                            cb.at[pl.ds(0, cnt)])

        def pgrp(gl, carry):
            lh = lane + gl * L
            for c in range(7):
                v = cbs[c][pl.ds(gl * L, L)]
                plsc.store_scatter(packp, [lh, _spl_i(c)], v)
            return carry
        lax.fori_loop(0, cnt // L, pgrp, 0)
        pltpu.sync_copy(packp.at[pl.ds(0, cnt)],
                        ptab_out.at[pl.ds(pbase + hoff, cnt)])


_pack_kernel = pl.kernel(
    _pack_body,
    out_type=(jax.ShapeDtypeStruct((N_NODES, ROW), _f32),
              jax.ShapeDtypeStruct((N_TYPES ** 4, ROW), _f32)),
    mesh=plsc.VectorSubcoreMesh(core_axis_name="c", subcore_axis_name="s"),
    compiler_params=pltpu.CompilerParams(
        needs_layout_passes=False, use_tc_tiling_on_sc=False),
    scratch_types=[
        pltpu.VMEM((PK_N_CNT * 3,), _f32),
        pltpu.VMEM((PK_N_CNT,), _i32),
        pltpu.VMEM((PK_N_CNT // 2, ROW), _f32),
    ] + [pltpu.VMEM((PK_P_HALF[0],), _f32)] * 7
    + [
        pltpu.VMEM((PK_P_HALF[0], ROW), _f32),
        pltpu.SemaphoreType.DMA,
    ],
)


def _prep_body(t0, k0, t1, k1, t2, k2, csum, a0, b0, a1, b1, a2, b2):
    csum[...] = k0[...] + k1[...] + k2[...]
    a0[...] = k0[...] * jnp.cos(t0[...])
    b0[...] = k0[...] * jnp.sin(t0[...])
    a1[...] = k1[...] * jnp.cos(t1[...])
    b1[...] = k1[...] * jnp.sin(t1[...])
    a2[...] = k2[...] * jnp.cos(t2[...])
    b2[...] = k2[...] * jnp.sin(t2[...])


def _finish_body(x, o):
    o[...] = jnp.sum(x[...], axis=0)


def kernel(pos, mapping, mapping_batch, atom_types,
           theta_0, k_0, theta_1, k_1, theta_2, k_2):
    ntab = N_TYPES ** 4
    shape2d = (ntab // 128, 128)
    tabs = [a.reshape(shape2d) for a in
            (theta_0, k_0, theta_1, k_1, theta_2, k_2)]
    coef = pl.pallas_call(
        _prep_body,
        out_shape=[jax.ShapeDtypeStruct(shape2d, _f32)] * 7,
    )(*tabs)

    nodes, ptab = _pack_kernel(
        pos.reshape(-1), atom_types, *[c.reshape(-1) for c in coef])

    mpad = [jnp.pad(mapping[j], (0, PADN - N_DIH)) for j in range(4)]
    bpad = jnp.pad(mapping_batch, (0, PADN - N_DIH),
                   constant_values=N_BATCH)
    part = _sc_kernel(nodes, mpad[0], mpad[1], mpad[2], mpad[3], bpad, ptab)

    eng = pl.pallas_call(
        _finish_body,
        out_shape=jax.ShapeDtypeStruct((ACC_SLOTS // 128, 128), _f32),
    )(part.reshape(NW, ACC_SLOTS // 128, 128))
    return eng.reshape(ACC_SLOTS)[:N_BATCH]
